# baseline (speedup 1.0000x reference)
import functools

import jax
import jax.numpy as jnp
from jax import lax
from jax.experimental import pallas as pl
from jax.experimental.pallas import tpu as pltpu
from jax.experimental.pallas import tpu_sc as plsc

N = 320000
D = 128
S = 10000
H1 = 32

CHUNK = 128
NUM_SC = 2
NTILES = 16
NW = NUM_SC * NTILES
CH_TILE = 80
NCH_PAD = NW * CH_TILE
N_PAD = NCH_PAD * CHUNK
NBUF = 3
ACC_ROWS = S
SEG_PER_TILE = 624

MM_BLK = 8000
MM_GRID = 41


def _tc_project(h, W1):
    def body(h_ref, w1_ref, g_ref):
        val = lax.dot_general(
            h_ref[...], w1_ref[...], (((1,), (0,)), ((), ())),
            precision=lax.Precision.HIGHEST,
            preferred_element_type=jnp.float32)
        pad = pl.program_id(0) >= N // MM_BLK
        g_ref[...] = jnp.where(pad, jnp.zeros_like(val), val)

    last = N // MM_BLK - 1
    return pl.pallas_call(
        body,
        grid=(MM_GRID,),
        in_specs=[
            pl.BlockSpec((MM_BLK, D), lambda i: (jnp.minimum(i, last), 0)),
            pl.BlockSpec((D, H1), lambda i: (0, 0)),
        ],
        out_specs=pl.BlockSpec((MM_BLK, H1), lambda i: (i, 0)),
        out_shape=jax.ShapeDtypeStruct((MM_GRID * MM_BLK, H1), jnp.float32),
    )(h, W1)


def _sc_segment_sum(g, ids_pad):
    mesh = plsc.VectorSubcoreMesh(core_axis_name="c", subcore_axis_name="s")

    @functools.partial(
        pl.kernel,
        out_type=jax.ShapeDtypeStruct((NUM_SC * S, H1), jnp.float32),
        mesh=mesh,
        scratch_types=[
            pltpu.VMEM((CHUNK, H1), jnp.float32),
            pltpu.VMEM((CHUNK, H1), jnp.float32),
            pltpu.VMEM((CHUNK, H1), jnp.float32),
            pltpu.VMEM((NBUF, CHUNK), jnp.int32),
            pltpu.VMEM_SHARED((ACC_ROWS, H1), jnp.float32),
            pltpu.SemaphoreType.DMA,
            pltpu.SemaphoreType.DMA,
            pltpu.SemaphoreType.DMA,
            pltpu.SemaphoreType.DMA,
            pltpu.SemaphoreType.DMA,
            pltpu.SemaphoreType.DMA,
        ],
    )
    def seg_sum(g_hbm, b_hbm, out_hbm, g0, g1, g2, ids3, acc,
                ld0, ld1, ld2, sc0, sc1, sc2):
        c = lax.axis_index("c")
        s = lax.axis_index("s")
        chunk0 = (c * NTILES + s) * CH_TILE

        gbuf = (g0, g1, g2)
        ld = (ld0, ld1, ld2)
        sc = (sc0, sc1, sc2)

        z16 = jnp.zeros((16,), jnp.float32)

        @pl.loop(0, CHUNK)
        def _(r):
            @pl.loop(0, H1 // 16)
            def _(gi):
                g0[r, pl.ds(gi * 16, 16)] = z16

        for z in range(SEG_PER_TILE // CHUNK):
            pltpu.sync_copy(
                g0, acc.at[pl.ds(s * SEG_PER_TILE + z * CHUNK, CHUNK)])
        pltpu.sync_copy(
            g0, acc.at[pl.ds(s * SEG_PER_TILE + SEG_PER_TILE - CHUNK, CHUNK)])

        @pl.when(s == NTILES - 1)
        def _():
            pltpu.sync_copy(g0, acc.at[pl.ds(ACC_ROWS - CHUNK, CHUNK)])

        plsc.subcore_barrier()

        def load(b, k):
            row = (chunk0 + k) * CHUNK
            pltpu.async_copy(g_hbm.at[pl.ds(row, CHUNK)], gbuf[b], ld[b])
            pltpu.async_copy(b_hbm.at[pl.ds(row, CHUNK)], ids3.at[b], ld[b])

        def wait_ld(b):
            pltpu.make_async_copy(
                g_hbm.at[pl.ds(0, CHUNK)], gbuf[b], ld[b]).wait()
            pltpu.make_async_copy(
                b_hbm.at[pl.ds(0, CHUNK)], ids3.at[b], ld[b]).wait()

        def fire(b):
            pltpu.async_copy(gbuf[b], acc.at[ids3.at[b]], sc[b], add=True)

        def drain(b):
            pltpu.make_async_copy(
                gbuf[b], acc.at[ids3.at[b]], sc[b]).wait()

        for b in range(NBUF):
            load(b, b)
        for k in range(CH_TILE):
            b = k % NBUF
            wait_ld(b)
            fire(b)
            if k >= NBUF - 1:
                drain((k + 1) % NBUF)
                if k + 1 < CH_TILE:
                    load((k + 1) % NBUF, k + 1)
        drain((CH_TILE - 2) % NBUF)
        drain((CH_TILE - 1) % NBUF)

        plsc.subcore_barrier()

        pltpu.sync_copy(
            acc.at[pl.ds(s * SEG_PER_TILE, SEG_PER_TILE)],
            out_hbm.at[pl.ds(c * S + s * SEG_PER_TILE, SEG_PER_TILE)])

        @pl.when(s == NTILES - 1)
        def _():
            pltpu.sync_copy(
                acc.at[pl.ds(NTILES * SEG_PER_TILE,
                             S - NTILES * SEG_PER_TILE)],
                out_hbm.at[pl.ds(c * S + NTILES * SEG_PER_TILE,
                                 S - NTILES * SEG_PER_TILE)])

    return seg_sum(g, ids_pad)


def _ssp(x):
    return jnp.maximum(x, 0.0) + jnp.log1p(jnp.exp(-jnp.abs(x))) \
        - jnp.log(2.0).astype(jnp.float32)


def _tc_tail(partials, b1r, W2, b2r):
    def body(p0_ref, p1_ref, b1_ref, w2_ref, b2_ref, o_ref):
        pooled = p0_ref[...] + p1_ref[...]
        t = _ssp(pooled + b1_ref[...])
        u = lax.dot_general(t, w2_ref[...], (((1,), (0,)), ((), ())),
                            precision=lax.Precision.HIGHEST,
                            preferred_element_type=jnp.float32)
        o_ref[...] = _ssp(u + b2_ref[...])

    return pl.pallas_call(
        body,
        grid=(1,),
        in_specs=[
            pl.BlockSpec((S, H1), lambda i: (0, 0)),
            pl.BlockSpec((S, H1), lambda i: (1, 0)),
            pl.BlockSpec((1, H1), lambda i: (0, 0)),
            pl.BlockSpec((H1, 1), lambda i: (0, 0)),
            pl.BlockSpec((1, 1), lambda i: (0, 0)),
        ],
        out_specs=pl.BlockSpec((S, 1), lambda i: (0, 0)),
        out_shape=jax.ShapeDtypeStruct((S, 1), jnp.float32),
    )(partials, partials, b1r, W2, b2r)


def kernel(h, batch, W1, b1, W2, b2):
    g = _tc_project(h, W1)
    ids_pad = jnp.concatenate(
        [batch.astype(jnp.int32), jnp.zeros((N_PAD - N,), jnp.int32)])
    partials = _sc_segment_sum(g, ids_pad)
    return _tc_tail(partials, b1.reshape(1, H1), W2, b2.reshape(1, 1))


# speedup vs baseline: 1.8645x; 1.8645x over previous
import functools

import jax
import jax.numpy as jnp
from jax import lax
from jax.experimental import pallas as pl
from jax.experimental.pallas import tpu as pltpu
from jax.experimental.pallas import tpu_sc as plsc

N = 320000
D = 128
S = 10000
H1 = 32

CHUNK = 128
NCH_TOTAL = N // CHUNK
NUM_SC = 2
NTILES = 16
NCH_SC = NCH_TOTAL // NUM_SC
BASE = NCH_SC // NTILES
EXTRA = NCH_SC - BASE * NTILES
SEG_PER_TILE = 624
ZROWS = 16


def _sc_segment_sum(h, batch_i32):
    mesh = plsc.VectorSubcoreMesh(core_axis_name="c", subcore_axis_name="s")

    @functools.partial(
        pl.kernel,
        out_type=jax.ShapeDtypeStruct((NUM_SC * S, D), jnp.float32),
        mesh=mesh,
        scratch_types=[
            pltpu.VMEM((CHUNK, D), jnp.float32),
            pltpu.VMEM((CHUNK, D), jnp.float32),
            pltpu.VMEM((CHUNK,), jnp.int32),
            pltpu.VMEM((CHUNK,), jnp.int32),
            pltpu.VMEM((ZROWS, D), jnp.float32),
            pltpu.VMEM_SHARED((S, D), jnp.float32),
            pltpu.SemaphoreType.DMA,
            pltpu.SemaphoreType.DMA,
            pltpu.SemaphoreType.DMA,
            pltpu.SemaphoreType.DMA,
        ],
    )
    def seg_sum(h_hbm, b_hbm, out_hbm, hA, hB, iA, iB, zb, acc,
                sAh, sBh, sAi, sBi):
        c = lax.axis_index("c")
        s = lax.axis_index("s")
        nch = BASE + jnp.where(s < EXTRA, 1, 0)
        chunk0 = c * NCH_SC + s * BASE + jnp.minimum(s, EXTRA)

        z16 = jnp.zeros((16,), jnp.float32)

        @pl.loop(0, ZROWS)
        def _(r):
            @pl.loop(0, D // 16)
            def _(g):
                zb[r, pl.ds(g * 16, 16)] = z16

        @pl.loop(0, SEG_PER_TILE // ZROWS)
        def _(k):
            pltpu.sync_copy(
                zb, acc.at[pl.ds(s * SEG_PER_TILE + k * ZROWS, ZROWS)])

        @pl.when(s == NTILES - 1)
        def _():
            pltpu.sync_copy(zb, acc.at[pl.ds(NTILES * SEG_PER_TILE, ZROWS)])

        plsc.subcore_barrier()

        def start(hbuf, ibuf, sh, si, ci):
            row = ci * CHUNK
            pltpu.async_copy(h_hbm.at[pl.ds(row, CHUNK)], hbuf, sh)
            pltpu.async_copy(b_hbm.at[pl.ds(row, CHUNK)], ibuf, si)

        def finish_and_scatter(hbuf, ibuf, sh, si):
            pltpu.make_async_copy(h_hbm.at[pl.ds(0, CHUNK)], hbuf, sh).wait()
            pltpu.make_async_copy(b_hbm.at[pl.ds(0, CHUNK)], ibuf, si).wait()
            pltpu.sync_copy(hbuf, acc.at[ibuf], add=True)

        start(hA, iA, sAh, sAi, chunk0)
        start(hB, iB, sBh, sBi, chunk0 + 1)

        @pl.loop(0, BASE // 2)
        def _(p):
            finish_and_scatter(hA, iA, sAh, sAi)

            @pl.when(2 * p + 2 < nch)
            def _():
                start(hA, iA, sAh, sAi, chunk0 + 2 * p + 2)

            finish_and_scatter(hB, iB, sBh, sBi)

            @pl.when(2 * p + 3 < nch)
            def _():
                start(hB, iB, sBh, sBi, chunk0 + 2 * p + 3)

        @pl.when(nch > BASE)
        def _():
            finish_and_scatter(hA, iA, sAh, sAi)

        plsc.subcore_barrier()

        pltpu.sync_copy(
            acc.at[pl.ds(s * SEG_PER_TILE, SEG_PER_TILE)],
            out_hbm.at[pl.ds(c * S + s * SEG_PER_TILE, SEG_PER_TILE)])

        @pl.when(s == NTILES - 1)
        def _():
            pltpu.sync_copy(
                acc.at[pl.ds(NTILES * SEG_PER_TILE, ZROWS)],
                out_hbm.at[pl.ds(c * S + NTILES * SEG_PER_TILE, ZROWS)])

    return seg_sum(h, batch_i32)


def _ssp(x):
    return jnp.maximum(x, 0.0) + jnp.log1p(jnp.exp(-jnp.abs(x))) \
        - jnp.log(2.0).astype(jnp.float32)


def _tc_tail(partials, W1, b1r, W2, b2r):
    BLK = 1000
    grid = S // BLK

    def body(p0_ref, p1_ref, w1_ref, b1_ref, w2_ref, b2_ref, o_ref):
        pooled = p0_ref[...] + p1_ref[...]
        t = lax.dot_general(pooled, w1_ref[...], (((1,), (0,)), ((), ())),
                            precision=lax.Precision.HIGHEST,
                            preferred_element_type=jnp.float32)
        t = _ssp(t + b1_ref[...])
        u = lax.dot_general(t, w2_ref[...], (((1,), (0,)), ((), ())),
                            precision=lax.Precision.HIGHEST,
                            preferred_element_type=jnp.float32)
        o_ref[...] = _ssp(u + b2_ref[...])

    return pl.pallas_call(
        body,
        grid=(grid,),
        in_specs=[
            pl.BlockSpec((BLK, D), lambda i: (i, 0)),
            pl.BlockSpec((BLK, D), lambda i: (i + grid, 0)),
            pl.BlockSpec((D, H1), lambda i: (0, 0)),
            pl.BlockSpec((1, H1), lambda i: (0, 0)),
            pl.BlockSpec((H1, 1), lambda i: (0, 0)),
            pl.BlockSpec((1, 1), lambda i: (0, 0)),
        ],
        out_specs=pl.BlockSpec((BLK, 1), lambda i: (i, 0)),
        out_shape=jax.ShapeDtypeStruct((S, 1), jnp.float32),
    )(partials, partials, W1, b1r, W2, b2r)


def kernel(h, batch, W1, b1, W2, b2):
    partials = _sc_segment_sum(h, batch.astype(jnp.int32))
    return _tc_tail(partials, W1, b1.reshape(1, H1), W2, b2.reshape(1, 1))
